# baseline (speedup 1.0000x reference)
import functools
import math

import jax
import jax.numpy as jnp
from jax import lax
from jax.experimental import pallas as pl
from jax.experimental.pallas import tpu as pltpu
from jax.experimental.pallas import tpu_sc as plsc

_EMB = 32
_SCALE = math.sqrt(_EMB)
_LANES = 16
_CHUNK = 128
_B = 4096
_L = 200
_BT = _B // _CHUNK
_LT = _L // 8
_VOCAB = 1000000
_RTP = (_VOCAB + _CHUNK - 1) // _CHUNK


@functools.lru_cache(maxsize=None)
def _build_convert():
    info = plsc.get_sparse_core_info()
    nw = info.num_cores * info.num_subcores
    n_slab = _RTP
    n_main = n_slab // nw
    mesh = plsc.VectorSubcoreMesh(core_axis_name="c", subcore_axis_name="s")

    @functools.partial(
        pl.kernel,
        mesh=mesh,
        out_type=jax.ShapeDtypeStruct((_RTP * _CHUNK, _EMB), jnp.float32),
        compiler_params=pltpu.CompilerParams(
            use_tc_tiling_on_sc=False, needs_layout_passes=False
        ),
        scratch_types=[
            pltpu.VMEM((4, 4, 8, _CHUNK), jnp.float32),
            pltpu.VMEM((4, _CHUNK, _EMB + 1), jnp.float32),
            pltpu.SemaphoreType.DMA,
            pltpu.SemaphoreType.DMA,
            pltpu.SemaphoreType.DMA,
            pltpu.SemaphoreType.DMA,
            pltpu.SemaphoreType.DMA,
            pltpu.SemaphoreType.DMA,
            pltpu.SemaphoreType.DMA,
            pltpu.SemaphoreType.DMA,
        ],
    )
    def conv_kernel(
        tab4_hbm, lin_hbm, nbuf, obuf, g0, g1, g2, g3, o0, o1, o2, o3
    ):
        wid = lax.axis_index("s") * info.num_cores + lax.axis_index("c")
        gsem = (g0, g1, g2, g3)
        osem = (o0, o1, o2, o3)
        iota16 = lax.iota(jnp.int32, _LANES)
        rows = [iota16 + j * _LANES for j in range(_CHUNK // _LANES)]

        def fire_in(k, b):
            rt = wid + k * nw
            for et in range(4):
                pltpu.async_copy(tab4_hbm.at[et, rt], nbuf.at[b, et], gsem[b])

        def wait_in(b):
            for et in range(4):
                pltpu.make_async_copy(
                    tab4_hbm.at[0, 0], nbuf.at[b, et], gsem[b]
                ).wait()

        def fire_out(k, b):
            rt = wid + k * nw
            pltpu.async_copy(
                obuf.at[b, pl.ds(0, _CHUNK), pl.ds(0, _EMB)],
                lin_hbm.at[pl.ds(rt * _CHUNK, _CHUNK), :],
                osem[b],
            )

        def wait_out(b):
            pltpu.make_async_copy(
                obuf.at[b, pl.ds(0, _CHUNK), pl.ds(0, _EMB)],
                lin_hbm.at[pl.ds(0, _CHUNK), :],
                osem[b],
            ).wait()

        def transpose(b):
            @plsc.parallel_loop(0, _EMB, unroll=4)
            def _(e):
                col = jnp.full((_LANES,), e, jnp.int32)
                et = e // 8
                es = e % 8
                for j in range(_CHUNK // _LANES):
                    v = nbuf[b, et, es, pl.ds(j * _LANES, _LANES)]
                    plsc.store_scatter(obuf.at[b], [rows[j], col], v)

        for b in range(4):
            fire_in(b, b)

        def quad_body(k4, carry):
            for b in range(4):
                k = k4 * 4 + b
                wait_in(b)

                @pl.when(k >= 4)
                def _():
                    wait_out(b)

                transpose(b)

                @pl.when(k + 4 < n_main)
                def _():
                    fire_in(k + 4, b)

                fire_out(k, b)
            return carry

        lax.fori_loop(0, n_main // 4, quad_body, 0)
        for b in range(4):
            wait_out(b)

        @pl.when(wid < n_slab - n_main * nw)
        def _():
            rt = n_main * nw + wid
            for et in range(4):
                pltpu.sync_copy(tab4_hbm.at[et, rt], nbuf.at[0, et])
            transpose(0)
            pltpu.sync_copy(
                obuf.at[0, pl.ds(0, _CHUNK), pl.ds(0, _EMB)],
                lin_hbm.at[pl.ds(rt * _CHUNK, _CHUNK), :],
            )

    return conv_kernel


@functools.lru_cache(maxsize=None)
def _build_lookup():
    n_tokens = _B * _L
    info = plsc.get_sparse_core_info()
    nw = info.num_cores * info.num_subcores
    per_w = n_tokens // nw
    n_chunks = per_w // _CHUNK
    mesh = plsc.VectorSubcoreMesh(core_axis_name="c", subcore_axis_name="s")

    @functools.partial(
        pl.kernel,
        mesh=mesh,
        out_type=jax.ShapeDtypeStruct((_L, 4, _BT, 8, _CHUNK), jnp.float32),
        compiler_params=pltpu.CompilerParams(
            use_tc_tiling_on_sc=False, needs_layout_passes=False
        ),
        scratch_types=[
            pltpu.VMEM((per_w,), jnp.int32),
            pltpu.VMEM((4, _CHUNK, _EMB), jnp.float32),
            pltpu.VMEM((4, _EMB, _CHUNK + 1), jnp.float32),
            pltpu.SemaphoreType.DMA,
            pltpu.SemaphoreType.DMA,
            pltpu.SemaphoreType.DMA,
            pltpu.SemaphoreType.DMA,
            pltpu.SemaphoreType.DMA,
            pltpu.SemaphoreType.DMA,
            pltpu.SemaphoreType.DMA,
            pltpu.SemaphoreType.DMA,
        ],
    )
    def emb_kernel(
        tok_hbm, table_hbm, out_hbm, idx_v, gbuf, tbuf,
        g0, g1, g2, g3, o0, o1, o2, o3
    ):
        wid = lax.axis_index("s") * info.num_cores + lax.axis_index("c")
        base = wid * per_w
        gsem = (g0, g1, g2, g3)
        osem = (o0, o1, o2, o3)
        pltpu.sync_copy(tok_hbm.at[pl.ds(base, per_w)], idx_v)

        def fire_gather(c, b):
            pltpu.async_copy(
                table_hbm.at[idx_v.at[pl.ds(c * _CHUNK, _CHUNK)]],
                gbuf.at[b],
                gsem[b],
            )

        def wait_gather(b):
            pltpu.make_async_copy(
                table_hbm.at[pl.ds(0, _CHUNK)], gbuf.at[b], gsem[b]
            ).wait()

        def chunk_coords(c):
            k = base // _CHUNK + c
            lt = k // (_BT * 8)
            r1 = k % (_BT * 8)
            bt = r1 // 8
            ls = r1 % 8
            return lt * 8 + ls, bt

        def fire_out(c, b):
            l, bt = chunk_coords(c)
            for et in range(4):
                pltpu.async_copy(
                    tbuf.at[b, pl.ds(et * 8, 8), pl.ds(0, _CHUNK)],
                    out_hbm.at[l, et, bt],
                    osem[b],
                )

        def wait_out(b):
            for et in range(4):
                pltpu.make_async_copy(
                    tbuf.at[b, pl.ds(et * 8, 8), pl.ds(0, _CHUNK)],
                    out_hbm.at[0, 0, 0],
                    osem[b],
                ).wait()

        iota16 = lax.iota(jnp.int32, _LANES)

        def transpose(b):
            @plsc.parallel_loop(0, _CHUNK, unroll=8)
            def _(bl):
                col = jnp.full((_LANES,), bl, jnp.int32)
                for h in range(2):
                    v = gbuf[b, bl, pl.ds(h * _LANES, _LANES)] * _SCALE
                    plsc.store_scatter(
                        tbuf.at[b], [iota16 + h * _LANES, col], v
                    )

        for b in range(4):
            fire_gather(b, b)

        def quad_body(c4, carry):
            for b in range(4):
                c = c4 * 4 + b
                wait_gather(b)

                @pl.when(c >= 4)
                def _():
                    wait_out(b)

                transpose(b)

                @pl.when(c + 4 < n_chunks)
                def _():
                    fire_gather(c + 4, b)

                fire_out(c, b)
            return carry

        lax.fori_loop(0, n_chunks // 4, quad_body, 0)
        for b in range(4):
            wait_out(b)

    return emb_kernel


def kernel(tokens, table):
    flat = (
        tokens.astype(jnp.int32)
        .reshape(_BT, _CHUNK, _LT, 8)
        .transpose(2, 0, 3, 1)
        .reshape(-1)
    )
    tabp = jnp.pad(table, ((0, _RTP * _CHUNK - _VOCAB), (0, 0)))
    tab4 = tabp.reshape(_RTP, _CHUNK, 4, 8).transpose(2, 0, 3, 1)
    lin = _build_convert()(tab4)
    out5 = _build_lookup()(flat, lin)
    return out5.transpose(2, 4, 0, 1, 3).reshape(_B, _L, _EMB)


# speedup vs baseline: 1.0020x; 1.0020x over previous
import functools
import math

import jax
import jax.numpy as jnp
from jax import lax
from jax.experimental import pallas as pl
from jax.experimental.pallas import tpu as pltpu
from jax.experimental.pallas import tpu_sc as plsc

_EMB = 32
_SCALE = math.sqrt(_EMB)
_LANES = 16
_CHUNK = 128
_B = 4096
_L = 200
_BT = _B // _CHUNK
_LT = _L // 8
_VOCAB = 1000000
_RTP = (_VOCAB + _CHUNK - 1) // _CHUNK


@functools.lru_cache(maxsize=None)
def _build_convert():
    info = plsc.get_sparse_core_info()
    nw = info.num_cores * info.num_subcores
    n_slab = _RTP
    n_main = n_slab // nw
    mesh = plsc.VectorSubcoreMesh(core_axis_name="c", subcore_axis_name="s")

    @functools.partial(
        pl.kernel,
        mesh=mesh,
        out_type=jax.ShapeDtypeStruct((_RTP * _CHUNK, _EMB), jnp.float32),
        compiler_params=pltpu.CompilerParams(
            use_tc_tiling_on_sc=False, needs_layout_passes=False
        ),
        scratch_types=[
            pltpu.VMEM((2, 4, 8, _CHUNK), jnp.float32),
            pltpu.VMEM((2, _CHUNK, _EMB + 1), jnp.float32),
            pltpu.SemaphoreType.DMA,
            pltpu.SemaphoreType.DMA,
            pltpu.SemaphoreType.DMA,
            pltpu.SemaphoreType.DMA,
        ],
    )
    def conv_kernel(tab4_hbm, lin_hbm, nbuf, obuf, g0, g1, o0, o1):
        wid = lax.axis_index("s") * info.num_cores + lax.axis_index("c")
        gsem = (g0, g1)
        osem = (o0, o1)
        iota16 = lax.iota(jnp.int32, _LANES)
        rows = [iota16 + j * _LANES for j in range(_CHUNK // _LANES)]

        def fire_in(k, b):
            rt = wid + k * nw
            for et in range(4):
                pltpu.async_copy(tab4_hbm.at[et, rt], nbuf.at[b, et], gsem[b])

        def wait_in(b):
            for et in range(4):
                pltpu.make_async_copy(
                    tab4_hbm.at[0, 0], nbuf.at[b, et], gsem[b]
                ).wait()

        def fire_out(k, b):
            rt = wid + k * nw
            pltpu.async_copy(
                obuf.at[b, pl.ds(0, _CHUNK), pl.ds(0, _EMB)],
                lin_hbm.at[pl.ds(rt * _CHUNK, _CHUNK), :],
                osem[b],
            )

        def wait_out(b):
            pltpu.make_async_copy(
                obuf.at[b, pl.ds(0, _CHUNK), pl.ds(0, _EMB)],
                lin_hbm.at[pl.ds(0, _CHUNK), :],
                osem[b],
            ).wait()

        def transpose(b):
            @plsc.parallel_loop(0, _EMB, unroll=4)
            def _(e):
                col = jnp.full((_LANES,), e, jnp.int32)
                et = e // 8
                es = e % 8
                for j in range(_CHUNK // _LANES):
                    v = nbuf[b, et, es, pl.ds(j * _LANES, _LANES)]
                    plsc.store_scatter(obuf.at[b], [rows[j], col], v)

        fire_in(0, 0)
        fire_in(1, 1)

        def pair_body(k2, carry):
            for b in range(2):
                k = k2 * 2 + b
                wait_in(b)

                @pl.when(k >= 2)
                def _():
                    wait_out(b)

                transpose(b)

                @pl.when(k + 2 < n_main)
                def _():
                    fire_in(k + 2, b)

                fire_out(k, b)
            return carry

        lax.fori_loop(0, n_main // 2, pair_body, 0)
        wait_out(0)
        wait_out(1)

        @pl.when(wid < n_slab - n_main * nw)
        def _():
            rt = n_main * nw + wid
            for et in range(4):
                pltpu.sync_copy(tab4_hbm.at[et, rt], nbuf.at[0, et])
            transpose(0)
            pltpu.sync_copy(
                obuf.at[0, pl.ds(0, _CHUNK), pl.ds(0, _EMB)],
                lin_hbm.at[pl.ds(rt * _CHUNK, _CHUNK), :],
            )

    return conv_kernel


@functools.lru_cache(maxsize=None)
def _build_lookup():
    n_tokens = _B * _L
    info = plsc.get_sparse_core_info()
    nw = info.num_cores * info.num_subcores
    per_w = n_tokens // nw
    n_chunks = per_w // _CHUNK
    mesh = plsc.VectorSubcoreMesh(core_axis_name="c", subcore_axis_name="s")

    @functools.partial(
        pl.kernel,
        mesh=mesh,
        out_type=jax.ShapeDtypeStruct((_L, 4, _BT, 8, _CHUNK), jnp.float32),
        compiler_params=pltpu.CompilerParams(
            use_tc_tiling_on_sc=False, needs_layout_passes=False
        ),
        scratch_types=[
            pltpu.VMEM((per_w,), jnp.int32),
            pltpu.VMEM((4, _CHUNK, _EMB), jnp.float32),
            pltpu.VMEM((4, _EMB, _CHUNK + 1), jnp.float32),
            pltpu.SemaphoreType.DMA,
            pltpu.SemaphoreType.DMA,
            pltpu.SemaphoreType.DMA,
            pltpu.SemaphoreType.DMA,
            pltpu.SemaphoreType.DMA,
            pltpu.SemaphoreType.DMA,
            pltpu.SemaphoreType.DMA,
            pltpu.SemaphoreType.DMA,
        ],
    )
    def emb_kernel(
        tok_hbm, table_hbm, out_hbm, idx_v, gbuf, tbuf,
        g0, g1, g2, g3, o0, o1, o2, o3
    ):
        wid = lax.axis_index("s") * info.num_cores + lax.axis_index("c")
        base = wid * per_w
        gsem = (g0, g1, g2, g3)
        osem = (o0, o1, o2, o3)
        pltpu.sync_copy(tok_hbm.at[pl.ds(base, per_w)], idx_v)

        def fire_gather(c, b):
            pltpu.async_copy(
                table_hbm.at[idx_v.at[pl.ds(c * _CHUNK, _CHUNK)]],
                gbuf.at[b],
                gsem[b],
            )

        def wait_gather(b):
            pltpu.make_async_copy(
                table_hbm.at[pl.ds(0, _CHUNK)], gbuf.at[b], gsem[b]
            ).wait()

        def chunk_coords(c):
            k = base // _CHUNK + c
            lt = k // (_BT * 8)
            r1 = k % (_BT * 8)
            bt = r1 // 8
            ls = r1 % 8
            return lt * 8 + ls, bt

        def fire_out(c, b):
            l, bt = chunk_coords(c)
            for et in range(4):
                pltpu.async_copy(
                    tbuf.at[b, pl.ds(et * 8, 8), pl.ds(0, _CHUNK)],
                    out_hbm.at[l, et, bt],
                    osem[b],
                )

        def wait_out(b):
            for et in range(4):
                pltpu.make_async_copy(
                    tbuf.at[b, pl.ds(et * 8, 8), pl.ds(0, _CHUNK)],
                    out_hbm.at[0, 0, 0],
                    osem[b],
                ).wait()

        iota16 = lax.iota(jnp.int32, _LANES)

        def transpose(b):
            @plsc.parallel_loop(0, _CHUNK, unroll=8)
            def _(bl):
                col = jnp.full((_LANES,), bl, jnp.int32)
                for h in range(2):
                    v = gbuf[b, bl, pl.ds(h * _LANES, _LANES)] * _SCALE
                    plsc.store_scatter(
                        tbuf.at[b], [iota16 + h * _LANES, col], v
                    )

        for b in range(4):
            fire_gather(b, b)

        def quad_body(c4, carry):
            for b in range(4):
                c = c4 * 4 + b
                wait_gather(b)

                @pl.when(c >= 4)
                def _():
                    wait_out(b)

                transpose(b)

                @pl.when(c + 4 < n_chunks)
                def _():
                    fire_gather(c + 4, b)

                fire_out(c, b)
            return carry

        lax.fori_loop(0, n_chunks // 4, quad_body, 0)
        for b in range(4):
            wait_out(b)

    return emb_kernel


def kernel(tokens, table):
    flat = (
        tokens.astype(jnp.int32)
        .reshape(_BT, _CHUNK, _LT, 8)
        .transpose(2, 0, 3, 1)
        .reshape(-1)
    )
    tabp = jnp.pad(table, ((0, _RTP * _CHUNK - _VOCAB), (0, 0)))
    tab4 = tabp.reshape(_RTP, _CHUNK, 4, 8).transpose(2, 0, 3, 1)
    lin = _build_convert()(tab4)
    out5 = _build_lookup()(flat, lin)
    return out5.transpose(2, 4, 0, 1, 3).reshape(_B, _L, _EMB)
